# baseline (speedup 1.0000x reference)
import functools

import jax
import jax.numpy as jnp
from jax import lax
from jax.experimental import pallas as pl
from jax.experimental.pallas import tpu as pltpu
from jax.experimental.pallas import tpu_sc as plsc

N = 10000
E = 320000
F_IN = 128
DIM = 32
NCLS = 16
NGRAPH = 64
BN_EPS = 1e-5

PACK = 128 // DIM
NP = 10240
R4 = NP // PACK
RV = N // PACK

NCORES = 2
NSUB = 16
NWORKERS = NCORES * NSUB
CH = 125
TOTROWS = E // CH
KCH = 10
NMEGA = TOTROWS // (NWORKERS * KCH)
WS = NP // NSUB
EPW = E // NWORKERS
ZR = 160

_HIGH = lax.Precision.HIGHEST


def _segment_sum_sc(p_lin, ei3):
    mesh = plsc.VectorSubcoreMesh(
        core_axis_name="c", subcore_axis_name="s",
        num_cores=NCORES, num_subcores=NSUB)

    @functools.partial(
        pl.kernel,
        out_type=jax.ShapeDtypeStruct((NCORES, NP, DIM), jnp.float32),
        mesh=mesh,
        scratch_types=[
            pltpu.VMEM((EPW // CH, CH), jnp.int32),
            pltpu.VMEM((EPW // CH, CH), jnp.int32),
            pltpu.VMEM((2, KCH, CH, DIM), jnp.float32),
            pltpu.VMEM((ZR, DIM), jnp.float32),
            pltpu.VMEM_SHARED((NP, DIM), jnp.float32),
            pltpu.SemaphoreType.DMA,
            pltpu.SemaphoreType.DMA,
            pltpu.SemaphoreType.DMA,
            pltpu.SemaphoreType.DMA,
        ],
        compiler_params=pltpu.CompilerParams(use_tc_tiling_on_sc=False),
    )
    def seg_kernel(p_hbm, ei3_hbm, out_hbm,
                   srcbuf, dstbuf, rows, zbuf, acc,
                   gsem0, gsem1, ssem0, ssem1):
        cid = lax.axis_index("c")
        sid = lax.axis_index("s")
        wid = cid * NSUB + sid
        gsems = (gsem0, gsem1)
        ssems = (ssem0, ssem1)

        nrw = EPW // CH
        pltpu.sync_copy(ei3_hbm.at[0, pl.ds(wid * nrw, nrw)], srcbuf)
        pltpu.sync_copy(ei3_hbm.at[1, pl.ds(wid * nrw, nrw)], dstbuf)

        @pl.when(cid == 0)
        def _():
            pltpu.sync_copy(p_hbm.at[pl.ds(sid * WS, WS)],
                            acc.at[pl.ds(sid * WS, WS)])

        @pl.when(cid == 1)
        def _():
            def zrow(r, carry):
                zbuf[r, pl.ds(0, 16)] = jnp.zeros((16,), jnp.float32)
                zbuf[r, pl.ds(16, 16)] = jnp.zeros((16,), jnp.float32)
                return carry
            lax.fori_loop(0, ZR, zrow, 0)
            for j in range(WS // ZR):
                pltpu.sync_copy(zbuf, acc.at[pl.ds(sid * WS + j * ZR, ZR)])
        plsc.subcore_barrier()

        def fire_gathers(m):
            b = m % 2
            for k in range(KCH):
                pltpu.async_copy(p_hbm.at[srcbuf.at[m * KCH + k]],
                                 rows.at[b, k], gsems[b])

        def drain_gathers(m):
            b = m % 2
            for k in range(KCH):
                pltpu.make_async_copy(p_hbm.at[srcbuf.at[m * KCH + k]],
                                      rows.at[b, k], gsems[b]).wait()

        def fire_scatters(m):
            b = m % 2
            for k in range(KCH):
                pltpu.async_copy(rows.at[b, k],
                                 acc.at[dstbuf.at[m * KCH + k]],
                                 ssems[b], add=True)

        def drain_scatters(m):
            b = m % 2
            for k in range(KCH):
                pltpu.make_async_copy(rows.at[b, k],
                                      acc.at[dstbuf.at[m * KCH + k]],
                                      ssems[b]).wait()

        def drain_fire(m):
            b = m % 2
            for k in range(KCH):
                pltpu.make_async_copy(p_hbm.at[srcbuf.at[m * KCH + k]],
                                      rows.at[b, k], gsems[b]).wait()
                pltpu.async_copy(rows.at[b, k],
                                 acc.at[dstbuf.at[m * KCH + k]],
                                 ssems[b], add=True)

        fire_gathers(0)
        for m in range(NMEGA):
            if m + 1 < NMEGA:
                if m >= 1:
                    drain_scatters(m - 1)
                fire_gathers(m + 1)
            drain_fire(m)
        drain_scatters(NMEGA - 2)
        drain_scatters(NMEGA - 1)

        plsc.subcore_barrier()
        pltpu.sync_copy(acc.at[pl.ds(sid * WS, WS)],
                        out_hbm.at[cid, pl.ds(sid * WS, WS)])

    return seg_kernel(p_lin, ei3)


BLK = 256
NBLK = R4 // BLK


def _proj_kernel(x_ref, w_ref, o_ref):
    xg = x_ref[...].reshape(BLK, PACK, F_IN)
    for c in range(PACK):
        o_ref[:, c * DIM:(c + 1) * DIM] = jnp.dot(
            xg[:, c, :], w_ref[...],
            preferred_element_type=jnp.float32, precision=_HIGH)


def _proj(x, w):
    return pl.pallas_call(
        _proj_kernel,
        grid=(NBLK,),
        in_specs=[pl.BlockSpec((PACK * BLK, F_IN), lambda i: (i, 0)),
                  pl.BlockSpec((F_IN, DIM), lambda i: (0, 0))],
        out_specs=pl.BlockSpec((BLK, 128), lambda i: (i, 0)),
        out_shape=jax.ShapeDtypeStruct((R4, 128), jnp.float32),
    )(x, w)


def _layer_head(part_ref, ba_ref, wb_ref, bb_ref, g_ref, bt_ref):
    q = part_ref[0] + part_ref[1] + ba_ref[...]
    r = jnp.maximum(q, 0.0)
    s = jnp.dot(r, wb_ref[...], preferred_element_type=jnp.float32,
                precision=_HIGH) + bb_ref[...]
    scale = g_ref[...] * lax.rsqrt(jnp.float32(1.0 + BN_EPS))
    return jnp.maximum(s, 0.0) * scale + bt_ref[...]


def _boundary_kernel(part_ref, ba_ref, wb_ref, bb_ref, g_ref, bt_ref,
                     wa_ref, o_ref):
    h = _layer_head(part_ref, ba_ref, wb_ref, bb_ref, g_ref, bt_ref)
    o_ref[...] = jnp.dot(h, wa_ref[...], preferred_element_type=jnp.float32,
                         precision=_HIGH)


def _boundary(part4, ba, wb4, bb, g, bt, wa4_next):
    vec = pl.BlockSpec((1, 128), lambda i: (0, 0))
    mat = pl.BlockSpec((128, 128), lambda i: (0, 0))
    return pl.pallas_call(
        _boundary_kernel,
        grid=(NBLK,),
        in_specs=[pl.BlockSpec((NCORES, BLK, 128), lambda i: (0, i, 0)),
                  vec, mat, vec, vec, vec, mat],
        out_specs=pl.BlockSpec((BLK, 128), lambda i: (i, 0)),
        out_shape=jax.ShapeDtypeStruct((R4, 128), jnp.float32),
    )(part4, ba, wb4, bb, g, bt, wa4_next)


def _final_kernel(part_ref, ba_ref, wb_ref, bb_ref, g_ref, bt_ref,
                  batch_ref, fw1_ref, fb1_ref, fw2_ref, fb2_ref, o_ref,
                  acc_ref):
    i = pl.program_id(0)
    h = _layer_head(part_ref, ba_ref, wb_ref, bb_ref, g_ref, bt_ref)
    rid = lax.broadcasted_iota(jnp.int32, (BLK, 1), 0) + i * BLK
    h = jnp.where(rid < RV, h, 0.0)

    @pl.when(i == 0)
    def _():
        acc_ref[...] = jnp.zeros_like(acc_ref)

    pooled = acc_ref[...]
    for c in range(PACK):
        oh = (lax.broadcasted_iota(jnp.int32, (NGRAPH, BLK), 0)
              == batch_ref[c]).astype(jnp.float32)
        pm = jnp.dot(oh, h, preferred_element_type=jnp.float32,
                     precision=_HIGH)
        pooled = pooled + pm[:, c * DIM:(c + 1) * DIM]
    acc_ref[...] = pooled

    @pl.when(i == NBLK - 1)
    def _():
        t = jnp.maximum(jnp.dot(pooled, fw1_ref[...],
                                preferred_element_type=jnp.float32,
                                precision=_HIGH) + fb1_ref[...], 0.0)
        o = jnp.dot(t, fw2_ref[...], preferred_element_type=jnp.float32,
                    precision=_HIGH) + fb2_ref[...]
        m = jnp.max(o, axis=-1, keepdims=True)
        lse = jnp.log(jnp.sum(jnp.exp(o - m), axis=-1, keepdims=True)) + m
        o_ref[...] = o - lse


def _final(part4, ba, wb4, bb, g, bt, batch_ct, fw1, fb1, fw2, fb2):
    vec = pl.BlockSpec((1, 128), lambda i: (0, 0))
    mat = pl.BlockSpec((128, 128), lambda i: (0, 0))
    return pl.pallas_call(
        _final_kernel,
        grid=(NBLK,),
        in_specs=[pl.BlockSpec((NCORES, BLK, 128), lambda i: (0, i, 0)),
                  vec, mat, vec, vec, vec,
                  pl.BlockSpec((PACK, BLK), lambda i: (0, i)),
                  pl.BlockSpec((DIM, DIM), lambda i: (0, 0)),
                  pl.BlockSpec((1, DIM), lambda i: (0, 0)),
                  pl.BlockSpec((DIM, NCLS), lambda i: (0, 0)),
                  pl.BlockSpec((1, NCLS), lambda i: (0, 0))],
        out_specs=pl.BlockSpec((NGRAPH, NCLS), lambda i: (0, 0)),
        out_shape=jax.ShapeDtypeStruct((NGRAPH, NCLS), jnp.float32),
        scratch_shapes=[pltpu.VMEM((NGRAPH, DIM), jnp.float32)],
    )(part4, ba, wb4, bb, g, bt, batch_ct, fw1, fb1, fw2, fb2)


def kernel(x, params, edge_index, batch):
    ei3 = edge_index.astype(jnp.int32).reshape(2, TOTROWS, CH)
    batch_ct = jnp.concatenate(
        [batch.astype(jnp.int32),
         jnp.full((NP - N,), -1, jnp.int32)]).reshape(R4, PACK).T

    eye4 = jnp.eye(PACK, dtype=jnp.float32)
    kron = lambda w: jnp.kron(eye4, w)
    tile = lambda v: jnp.tile(v, PACK).reshape(1, 128)

    p4 = _proj(x, params["w0a"])
    for i in range(5):
        part = _segment_sum_sc(p4.reshape(NP, DIM), ei3)
        part4 = part.reshape(NCORES, R4, 128)
        args = (part4, tile(params[f"b{i}a"]), kron(params[f"w{i}b"]),
                tile(params[f"b{i}b"]), tile(params[f"g{i}"]),
                tile(params[f"bt{i}"]))
        if i < 4:
            p4 = _boundary(*args, kron(params[f"w{i+1}a"]))
        else:
            out = _final(*args, batch_ct,
                         params["fw1"], params["fb1"].reshape(1, DIM),
                         params["fw2"], params["fb2"].reshape(1, NCLS))
    return out


# speedup vs baseline: 1.0604x; 1.0604x over previous
import functools

import jax
import jax.numpy as jnp
from jax import lax
from jax.experimental import pallas as pl
from jax.experimental.pallas import tpu as pltpu
from jax.experimental.pallas import tpu_sc as plsc

N = 10000
E = 320000
F_IN = 128
DIM = 32
NCLS = 16
NGRAPH = 64
BN_EPS = 1e-5

PACK = 128 // DIM
NP = 10240
R4 = NP // PACK
RV = N // PACK

NCORES = 2
NSUB = 16
NWORKERS = NCORES * NSUB
CH = 125
TOTROWS = E // CH
KCH = 10
NMEGA = TOTROWS // (NWORKERS * KCH)
WS = NP // NSUB
EPW = E // NWORKERS
ZR = 160

_HIGH = lax.Precision.HIGHEST


def _segment_sum_sc(p_lin, ei3):
    mesh = plsc.VectorSubcoreMesh(
        core_axis_name="c", subcore_axis_name="s",
        num_cores=NCORES, num_subcores=NSUB)

    @functools.partial(
        pl.kernel,
        out_type=jax.ShapeDtypeStruct((NCORES, NP, DIM), jnp.float32),
        mesh=mesh,
        scratch_types=[
            pltpu.VMEM((EPW // CH, CH), jnp.int32),
            pltpu.VMEM((EPW // CH, CH), jnp.int32),
            pltpu.VMEM((2, KCH, CH, DIM), jnp.float32),
            pltpu.VMEM((ZR, DIM), jnp.float32),
            pltpu.VMEM_SHARED((NP, DIM), jnp.float32),
            pltpu.SemaphoreType.DMA,
            pltpu.SemaphoreType.DMA,
            pltpu.SemaphoreType.DMA,
            pltpu.SemaphoreType.DMA,
        ],
        compiler_params=pltpu.CompilerParams(use_tc_tiling_on_sc=False),
    )
    def seg_kernel(p_hbm, ei3_hbm, out_hbm,
                   srcbuf, dstbuf, rows, zbuf, acc,
                   gsem0, gsem1, ssem0, ssem1):
        cid = lax.axis_index("c")
        sid = lax.axis_index("s")
        wid = cid * NSUB + sid
        gsems = (gsem0, gsem1)
        ssems = (ssem0, ssem1)

        nrw = EPW // CH
        pltpu.sync_copy(ei3_hbm.at[0, pl.ds(wid * nrw, nrw)], srcbuf)
        pltpu.sync_copy(ei3_hbm.at[1, pl.ds(wid * nrw, nrw)], dstbuf)

        @pl.when(cid == 0)
        def _():
            pltpu.sync_copy(p_hbm.at[pl.ds(sid * WS, WS)],
                            acc.at[pl.ds(sid * WS, WS)])

        @pl.when(cid == 1)
        def _():
            def zrow(r, carry):
                zbuf[r, pl.ds(0, 16)] = jnp.zeros((16,), jnp.float32)
                zbuf[r, pl.ds(16, 16)] = jnp.zeros((16,), jnp.float32)
                return carry
            lax.fori_loop(0, ZR, zrow, 0)
            for j in range(WS // ZR):
                pltpu.sync_copy(zbuf, acc.at[pl.ds(sid * WS + j * ZR, ZR)])
        plsc.subcore_barrier()

        def fire_gathers(m):
            b = m % 2
            for k in range(KCH):
                pltpu.async_copy(p_hbm.at[srcbuf.at[m * KCH + k]],
                                 rows.at[b, k], gsems[b])

        def drain_gathers(m):
            b = m % 2
            for k in range(KCH):
                pltpu.make_async_copy(p_hbm.at[srcbuf.at[m * KCH + k]],
                                      rows.at[b, k], gsems[b]).wait()

        def fire_scatters(m):
            b = m % 2
            for k in range(KCH):
                pltpu.async_copy(rows.at[b, k],
                                 acc.at[dstbuf.at[m * KCH + k]],
                                 ssems[b], add=True)

        def drain_scatters(m):
            b = m % 2
            for k in range(KCH):
                pltpu.make_async_copy(rows.at[b, k],
                                      acc.at[dstbuf.at[m * KCH + k]],
                                      ssems[b]).wait()

        def drain_fire(m):
            b = m % 2
            for k in range(KCH):
                pltpu.make_async_copy(p_hbm.at[srcbuf.at[m * KCH + k]],
                                      rows.at[b, k], gsems[b]).wait()
                pltpu.async_copy(rows.at[b, k],
                                 acc.at[dstbuf.at[m * KCH + k]],
                                 ssems[b], add=True)

        fire_gathers(0)
        for m in range(NMEGA):
            if m + 1 < NMEGA:
                if m >= 1:
                    drain_scatters(m - 1)
                fire_gathers(m + 1)
            drain_fire(m)
        drain_scatters(NMEGA - 2)
        drain_scatters(NMEGA - 1)

        plsc.subcore_barrier()
        pltpu.sync_copy(acc.at[pl.ds(sid * WS, WS)],
                        out_hbm.at[cid, pl.ds(sid * WS, WS)])

    return seg_kernel(p_lin, ei3)


BLK = 512
NBLK = R4 // BLK


def _proj_kernel(x_ref, w_ref, o_ref):
    xg = x_ref[...].reshape(BLK, PACK, F_IN)
    for c in range(PACK):
        o_ref[:, c * DIM:(c + 1) * DIM] = jnp.dot(
            xg[:, c, :], w_ref[...],
            preferred_element_type=jnp.float32, precision=_HIGH)


def _proj(x, w):
    return pl.pallas_call(
        _proj_kernel,
        grid=(NBLK,),
        in_specs=[pl.BlockSpec((PACK * BLK, F_IN), lambda i: (i, 0)),
                  pl.BlockSpec((F_IN, DIM), lambda i: (0, 0))],
        out_specs=pl.BlockSpec((BLK, 128), lambda i: (i, 0)),
        out_shape=jax.ShapeDtypeStruct((R4, 128), jnp.float32),
    )(x, w)


def _layer_head(part_ref, ba_ref, wb_ref, bb_ref, g_ref, bt_ref):
    q = part_ref[0] + part_ref[1] + ba_ref[...]
    r = jnp.maximum(q, 0.0)
    s = jnp.dot(r, wb_ref[...], preferred_element_type=jnp.float32,
                precision=_HIGH) + bb_ref[...]
    scale = g_ref[...] * lax.rsqrt(jnp.float32(1.0 + BN_EPS))
    return jnp.maximum(s, 0.0) * scale + bt_ref[...]


def _boundary_kernel(part_ref, ba_ref, wb_ref, bb_ref, g_ref, bt_ref,
                     wa_ref, o_ref):
    h = _layer_head(part_ref, ba_ref, wb_ref, bb_ref, g_ref, bt_ref)
    o_ref[...] = jnp.dot(h, wa_ref[...], preferred_element_type=jnp.float32,
                         precision=_HIGH)


def _boundary(part4, ba, wb4, bb, g, bt, wa4_next):
    vec = pl.BlockSpec((1, 128), lambda i: (0, 0))
    mat = pl.BlockSpec((128, 128), lambda i: (0, 0))
    return pl.pallas_call(
        _boundary_kernel,
        grid=(NBLK,),
        in_specs=[pl.BlockSpec((NCORES, BLK, 128), lambda i: (0, i, 0)),
                  vec, mat, vec, vec, vec, mat],
        out_specs=pl.BlockSpec((BLK, 128), lambda i: (i, 0)),
        out_shape=jax.ShapeDtypeStruct((R4, 128), jnp.float32),
    )(part4, ba, wb4, bb, g, bt, wa4_next)


def _final_kernel(part_ref, ba_ref, wb_ref, bb_ref, g_ref, bt_ref,
                  batch_ref, fw1_ref, fb1_ref, fw2_ref, fb2_ref, o_ref,
                  acc_ref):
    i = pl.program_id(0)
    h = _layer_head(part_ref, ba_ref, wb_ref, bb_ref, g_ref, bt_ref)
    rid = lax.broadcasted_iota(jnp.int32, (BLK, 1), 0) + i * BLK
    h = jnp.where(rid < RV, h, 0.0)

    @pl.when(i == 0)
    def _():
        acc_ref[...] = jnp.zeros_like(acc_ref)

    pooled = acc_ref[...]
    for c in range(PACK):
        oh = (lax.broadcasted_iota(jnp.int32, (NGRAPH, BLK), 0)
              == batch_ref[c]).astype(jnp.float32)
        pm = jnp.dot(oh, h, preferred_element_type=jnp.float32,
                     precision=_HIGH)
        pooled = pooled + pm[:, c * DIM:(c + 1) * DIM]
    acc_ref[...] = pooled

    @pl.when(i == NBLK - 1)
    def _():
        t = jnp.maximum(jnp.dot(pooled, fw1_ref[...],
                                preferred_element_type=jnp.float32,
                                precision=_HIGH) + fb1_ref[...], 0.0)
        o = jnp.dot(t, fw2_ref[...], preferred_element_type=jnp.float32,
                    precision=_HIGH) + fb2_ref[...]
        m = jnp.max(o, axis=-1, keepdims=True)
        lse = jnp.log(jnp.sum(jnp.exp(o - m), axis=-1, keepdims=True)) + m
        o_ref[...] = o - lse


def _final(part4, ba, wb4, bb, g, bt, batch_ct, fw1, fb1, fw2, fb2):
    vec = pl.BlockSpec((1, 128), lambda i: (0, 0))
    mat = pl.BlockSpec((128, 128), lambda i: (0, 0))
    return pl.pallas_call(
        _final_kernel,
        grid=(NBLK,),
        in_specs=[pl.BlockSpec((NCORES, BLK, 128), lambda i: (0, i, 0)),
                  vec, mat, vec, vec, vec,
                  pl.BlockSpec((PACK, BLK), lambda i: (0, i)),
                  pl.BlockSpec((DIM, DIM), lambda i: (0, 0)),
                  pl.BlockSpec((1, DIM), lambda i: (0, 0)),
                  pl.BlockSpec((DIM, NCLS), lambda i: (0, 0)),
                  pl.BlockSpec((1, NCLS), lambda i: (0, 0))],
        out_specs=pl.BlockSpec((NGRAPH, NCLS), lambda i: (0, 0)),
        out_shape=jax.ShapeDtypeStruct((NGRAPH, NCLS), jnp.float32),
        scratch_shapes=[pltpu.VMEM((NGRAPH, DIM), jnp.float32)],
    )(part4, ba, wb4, bb, g, bt, batch_ct, fw1, fb1, fw2, fb2)


def kernel(x, params, edge_index, batch):
    ei3 = edge_index.astype(jnp.int32).reshape(2, TOTROWS, CH)
    batch_ct = jnp.concatenate(
        [batch.astype(jnp.int32),
         jnp.full((NP - N,), -1, jnp.int32)]).reshape(R4, PACK).T

    eye4 = jnp.eye(PACK, dtype=jnp.float32)
    kron = lambda w: jnp.kron(eye4, w)
    tile = lambda v: jnp.tile(v, PACK).reshape(1, 128)

    p4 = _proj(x, params["w0a"])
    for i in range(5):
        part = _segment_sum_sc(p4.reshape(NP, DIM), ei3)
        part4 = part.reshape(NCORES, R4, 128)
        args = (part4, tile(params[f"b{i}a"]), kron(params[f"w{i}b"]),
                tile(params[f"b{i}b"]), tile(params[f"g{i}"]),
                tile(params[f"bt{i}"]))
        if i < 4:
            p4 = _boundary(*args, kron(params[f"w{i+1}a"]))
        else:
            out = _final(*args, batch_ct,
                         params["fw1"], params["fb1"].reshape(1, DIM),
                         params["fw2"], params["fb2"].reshape(1, NCLS))
    return out
